# baseline (speedup 1.0000x reference)
import functools

import jax
import jax.numpy as jnp
from jax import lax
from jax.experimental import pallas as pl
from jax.experimental.pallas import tpu as pltpu
from jax.experimental.pallas import tpu_sc as plsc

K = 10


def _tc_kernel(x_ref, wt_ref, b_ref, tags_ref, idx_ref, *, classes):
    x = x_ref[...]
    logits = jnp.dot(x, wt_ref[...], preferred_element_type=jnp.float32)
    logits = logits + b_ref[...]
    m = jnp.max(logits, axis=1, keepdims=True)
    e = jnp.exp(logits - m)
    s = jnp.sum(e, axis=1, keepdims=True)
    tags = e / s
    tags_ref[...] = tags

    idx_ref[...] = jnp.zeros(idx_ref.shape, jnp.int32)
    iota = jax.lax.broadcasted_iota(jnp.int32, tags.shape, 1)
    work = tags
    for k in range(K):
        mx = jnp.max(work, axis=1, keepdims=True)
        cand = jnp.where(work == mx, iota, classes)
        idxk = jnp.min(cand, axis=1, keepdims=True)
        idx_ref[:, k:k + 1] = idxk
        work = jnp.where(iota == idxk, -1.0, work)


def _make_sc_gather(n_rows, sem_dim, rows_per_worker, chunk):
    mesh = plsc.VectorSubcoreMesh(core_axis_name="c", subcore_axis_name="s")
    info = plsc.get_sparse_core_info()
    nc = info.num_cores

    @functools.partial(
        pl.kernel,
        mesh=mesh,
        out_type=jax.ShapeDtypeStruct((n_rows, sem_dim), jnp.float32),
        scratch_types=[
            pltpu.VMEM((chunk,), jnp.int32),
            pltpu.VMEM((chunk, sem_dim), jnp.float32),
            pltpu.SemaphoreType.DMA,
        ],
    )
    def gather(table_hbm, idx_hbm, out_hbm, idx_v, rows_v, sem):
        wid = lax.axis_index("s") * nc + lax.axis_index("c")
        base = wid * rows_per_worker

        def body(g, carry):
            off = base + g * chunk
            pltpu.sync_copy(idx_hbm.at[pl.ds(off, chunk)], idx_v)
            pltpu.async_copy(table_hbm.at[idx_v], rows_v, sem).wait()
            pltpu.sync_copy(rows_v, out_hbm.at[pl.ds(off, chunk)])
            return carry

        lax.fori_loop(0, rows_per_worker // chunk, body, 0)

    return gather


def kernel(avg_features, W, b, embed_table):
    B, fc_in = avg_features.shape
    classes, sem_dim = embed_table.shape
    tile = min(512, B)
    grid = (B // tile,)

    wt = W.T
    b2 = b.reshape(1, classes)

    tags, idx2 = pl.pallas_call(
        functools.partial(_tc_kernel, classes=classes),
        grid=grid,
        in_specs=[
            pl.BlockSpec((tile, fc_in), lambda i: (i, 0)),
            pl.BlockSpec((fc_in, classes), lambda i: (0, 0)),
            pl.BlockSpec((1, classes), lambda i: (0, 0)),
        ],
        out_specs=(
            pl.BlockSpec((tile, classes), lambda i: (i, 0)),
            pl.BlockSpec((tile, 16), lambda i: (i, 0)),
        ),
        out_shape=(
            jax.ShapeDtypeStruct((B, classes), jnp.float32),
            jax.ShapeDtypeStruct((B, 16), jnp.int32),
        ),
    )(avg_features, wt, b2)

    n_rows = B * 16
    idx1d = idx2.reshape(n_rows)
    rows_per_worker = n_rows // 32
    gather = _make_sc_gather(n_rows, sem_dim, rows_per_worker, 128)
    semf = gather(embed_table, idx1d)
    sem = semf.reshape(B, 16, sem_dim)[:, :K, :]
    return (tags, sem)


# speedup vs baseline: 10.5252x; 10.5252x over previous
import functools

import jax
import jax.numpy as jnp
from jax.experimental import pallas as pl

K = 10


def _fused_kernel(x_ref, wt_ref, b_ref, tab_ref, tags_ref, sem_ref, *, classes):
    x = x_ref[...]
    logits = jnp.dot(x, wt_ref[...], preferred_element_type=jnp.float32)
    logits = logits + b_ref[...]
    m = jnp.max(logits, axis=1, keepdims=True)
    e = jnp.exp(logits - m)
    s = jnp.sum(e, axis=1, keepdims=True)
    tags = e / s
    tags_ref[...] = tags

    iota = jax.lax.broadcasted_iota(jnp.int32, tags.shape, 1)
    tab = tab_ref[...]
    work = tags
    for k in range(K):
        mx = jnp.max(work, axis=1, keepdims=True)
        cand = jnp.where(work == mx, iota, classes)
        idxk = jnp.min(cand, axis=1, keepdims=True)
        hit = iota == idxk
        onehot = hit.astype(jnp.float32)
        row = jnp.dot(onehot, tab, preferred_element_type=jnp.float32)
        sem_ref[:, k, :] = row
        work = jnp.where(hit, -1.0, work)


def kernel(avg_features, W, b, embed_table):
    B, fc_in = avg_features.shape
    classes, sem_dim = embed_table.shape
    tile = min(512, B)
    grid = (B // tile,)

    wt = W.T
    b2 = b.reshape(1, classes)

    tags, sem = pl.pallas_call(
        functools.partial(_fused_kernel, classes=classes),
        grid=grid,
        in_specs=[
            pl.BlockSpec((tile, fc_in), lambda i: (i, 0)),
            pl.BlockSpec((fc_in, classes), lambda i: (0, 0)),
            pl.BlockSpec((1, classes), lambda i: (0, 0)),
            pl.BlockSpec((classes, sem_dim), lambda i: (0, 0)),
        ],
        out_specs=(
            pl.BlockSpec((tile, classes), lambda i: (i, 0)),
            pl.BlockSpec((tile, 16, sem_dim), lambda i: (i, 0, 0)),
        ),
        out_shape=(
            jax.ShapeDtypeStruct((B, classes), jnp.float32),
            jax.ShapeDtypeStruct((B, 16, sem_dim), jnp.float32),
        ),
    )(avg_features, wt, b2, embed_table)
    return (tags, sem[:, :K, :])


# speedup vs baseline: 10.5345x; 1.0009x over previous
import functools

import jax
import jax.numpy as jnp
from jax.experimental import pallas as pl
from jax.experimental.pallas import tpu as pltpu

K = 10


def _fused_kernel(x_ref, wt_ref, b_ref, tab_ref, tags_ref, sem_ref, *, classes):
    x = x_ref[...]
    logits = jnp.dot(x, wt_ref[...], preferred_element_type=jnp.float32)
    logits = logits + b_ref[...]
    m = jnp.max(logits, axis=1, keepdims=True)
    e = jnp.exp(logits - m)
    s = jnp.sum(e, axis=1, keepdims=True)
    tags = e / s
    tags_ref[...] = tags

    iota = jax.lax.broadcasted_iota(jnp.int32, tags.shape, 1)
    tab = tab_ref[...]
    work = tags
    for k in range(K):
        mx = jnp.max(work, axis=1, keepdims=True)
        cand = jnp.where(work == mx, iota, classes)
        idxk = jnp.min(cand, axis=1, keepdims=True)
        hit = iota == idxk
        onehot = hit.astype(jnp.float32)
        row = jnp.dot(onehot, tab, preferred_element_type=jnp.float32)
        sem_ref[:, k, :] = row
        work = jnp.where(hit, -1.0, work)


def kernel(avg_features, W, b, embed_table):
    B, fc_in = avg_features.shape
    classes, sem_dim = embed_table.shape
    tile = min(512, B)
    grid = (B // tile,)

    wt = W.T
    b2 = b.reshape(1, classes)

    tags, sem = pl.pallas_call(
        functools.partial(_fused_kernel, classes=classes),
        grid=grid,
        in_specs=[
            pl.BlockSpec((tile, fc_in), lambda i: (i, 0)),
            pl.BlockSpec((fc_in, classes), lambda i: (0, 0)),
            pl.BlockSpec((1, classes), lambda i: (0, 0)),
            pl.BlockSpec((classes, sem_dim), lambda i: (0, 0)),
        ],
        out_specs=(
            pl.BlockSpec((tile, classes), lambda i: (i, 0)),
            pl.BlockSpec((tile, 16, sem_dim), lambda i: (i, 0, 0)),
        ),
        out_shape=(
            jax.ShapeDtypeStruct((B, classes), jnp.float32),
            jax.ShapeDtypeStruct((B, 16, sem_dim), jnp.float32),
        ),
        compiler_params=pltpu.CompilerParams(
            dimension_semantics=("parallel",),
        ),
    )(avg_features, wt, b2, embed_table)
    return (tags, sem[:, :K, :])
